# baseline (speedup 1.0000x reference)
import functools

import jax
import jax.numpy as jnp
from jax.experimental import pallas as pl
from jax.experimental.pallas import tpu as pltpu


def _gather_embed_kernel(ids_ref, word_hbm, pos_ref, out_ref, buf, sems, *,
                         tile, n_inner):
    c = pl.program_id(0)
    j = pl.program_id(1)
    slot = j % 2
    t = c * n_inner + j

    def issue_rows(tile_idx, s):
        base = tile_idx * tile
        for r in range(tile):
            row = ids_ref[base + r]
            pltpu.make_async_copy(word_hbm.at[pl.ds(row, 1)],
                                  buf.at[s, pl.ds(r, 1)],
                                  sems.at[s]).start()

    @pl.when(j == 0)
    def _():
        issue_rows(t, 0)

    @pl.when(j + 1 < n_inner)
    def _():
        issue_rows(t + 1, 1 - slot)

    pltpu.make_async_copy(word_hbm.at[pl.ds(0, tile)], buf.at[slot],
                          sems.at[slot]).wait()

    out_ref[...] = buf[slot] + pos_ref[...]


def kernel(inputs, word_table, pos_table):
    B, L = inputs.shape
    V, D = word_table.shape
    S, D2 = pos_table.shape
    assert D == D2 and L <= S

    word_table = word_table.astype(jnp.float32)
    pos_table = pos_table.astype(jnp.float32)

    tile = L
    n_tiles = B
    n_cores = 2 if n_tiles % 2 == 0 else 1
    n_inner = n_tiles // n_cores

    ids = jnp.clip(inputs.astype(jnp.int32), 0, V - 1)
    ids_flat = ids.reshape(B * L)

    kernel_fn = functools.partial(_gather_embed_kernel, tile=tile,
                                  n_inner=n_inner)
    out_flat = pl.pallas_call(
        kernel_fn,
        out_shape=jax.ShapeDtypeStruct((B * L, D), jnp.float32),
        grid_spec=pltpu.PrefetchScalarGridSpec(
            num_scalar_prefetch=1,
            grid=(n_cores, n_inner),
            in_specs=[
                pl.BlockSpec(memory_space=pl.ANY),
                pl.BlockSpec((tile, D), lambda c, j, ids: (0, 0)),
            ],
            out_specs=pl.BlockSpec((tile, D),
                                   lambda c, j, ids: (c * n_inner + j, 0)),
            scratch_shapes=[
                pltpu.VMEM((2, tile, D), jnp.float32),
                pltpu.SemaphoreType.DMA((2,)),
            ],
        ),
        compiler_params=pltpu.CompilerParams(
            dimension_semantics=("parallel", "arbitrary"),
            vmem_limit_bytes=64 * 1024 * 1024),
    )(ids_flat, word_table, pos_table[:L])

    return out_flat.reshape(B, L, D)
